# Initial kernel scaffold; baseline (speedup 1.0000x reference)
import jax
import jax.numpy as jnp
from jax.experimental import pallas as pl


def kernel(x, edge_index_r1, edge_index_r2, edge_index_r3, params):
    raise NotImplementedError("write your pallas kernel here")


# speedup vs baseline: 2.3742x; 2.3742x over previous
import jax
import jax.numpy as jnp
from jax.experimental import pallas as pl

_N = 10000
_D = 128
_H = 128
_TOPK = 10
_RELS = ("r1", "r2", "r3")


def _rownorm(x):
    def body(x_ref, o_ref):
        v = x_ref[...]
        n = jnp.sqrt(jnp.sum(v * v, axis=1, keepdims=True))
        o_ref[...] = v / jnp.maximum(n, 1e-12)

    return pl.pallas_call(
        body, out_shape=jax.ShapeDtypeStruct(x.shape, x.dtype)
    )(x)


def _cheb_dense(t0, t1, t2, w0, w1, w2, b, hprev):
    def body(t0r, t1r, t2r, w0r, w1r, w2r, br, hpr, o_ref):
        acc = jnp.dot(t0r[...], w0r[...], preferred_element_type=jnp.float32)
        acc = acc + jnp.dot(t1r[...], w1r[...], preferred_element_type=jnp.float32)
        acc = acc + jnp.dot(t2r[...], w2r[...], preferred_element_type=jnp.float32)
        acc = acc + br[...] + hpr[...]
        o_ref[...] = jnp.maximum(acc, 0.0)

    return pl.pallas_call(
        body, out_shape=jax.ShapeDtypeStruct((t0.shape[0], w0.shape[1]), t0.dtype)
    )(t0, t1, t2, w0, w1, w2, b.reshape(1, -1), hprev)


def _fusion(e1, e2, e3, p):
    gw1 = p["gate_W1"]
    gb1 = p["gate_b1"].reshape(1, -1)
    gw2 = p["gate_W2"]
    gb2 = p["gate_b2"].reshape(1, 1)
    pw = p["proj_W"]
    pb = p["proj_b"].reshape(1, -1)
    cw1 = p["cls_W1"]
    cb1 = p["cls_b1"].reshape(1, -1)
    cw2 = p["cls_W2"]
    cb2 = p["cls_b2"].reshape(1, 1)
    aw = jnp.concatenate([p["aux"][r]["W"] for r in _RELS], axis=1)
    ab = jnp.concatenate([p["aux"][r]["b"] for r in _RELS]).reshape(1, 3)

    def body(e1r, e2r, e3r, gw1r, gb1r, gw2r, gb2r, pwr, pbr, cw1r, cb1r,
             cw2r, cb2r, awr, abr, o_ref):
        es = [e1r[...], e2r[...], e3r[...]]
        gs = []
        for e in es:
            h1 = jnp.maximum(
                jnp.dot(e, gw1r[...], preferred_element_type=jnp.float32)
                + gb1r[...], 0.0)
            gs.append(
                jnp.dot(h1, gw2r[...], preferred_element_type=jnp.float32)
                + gb2r[...])
        g = jnp.concatenate(gs, axis=1)
        m = jnp.max(g, axis=1, keepdims=True)
        ex = jnp.exp(g - m)
        a = ex / jnp.sum(ex, axis=1, keepdims=True)
        fused = a[:, 0:1] * es[0] + a[:, 1:2] * es[1] + a[:, 2:3] * es[2]
        h = jnp.maximum(
            jnp.dot(fused, pwr[...], preferred_element_type=jnp.float32)
            + pbr[...], 0.0)
        h2 = jnp.maximum(
            jnp.dot(h, cw1r[...], preferred_element_type=jnp.float32)
            + cb1r[...], 0.0)
        logit = jnp.dot(h2, cw2r[...], preferred_element_type=jnp.float32) + cb2r[...]
        aux = jnp.concatenate(
            [jnp.dot(e, awr[...], preferred_element_type=jnp.float32)[:, i:i + 1]
             for i, e in enumerate(es)], axis=1) + abr[...]
        o_ref[...] = jnp.concatenate([logit, aux], axis=1)

    out = pl.pallas_call(
        body, out_shape=jax.ShapeDtypeStruct((e1.shape[0], 4), e1.dtype)
    )(e1, e2, e3, gw1, gb1, gw2, gb2, pw, pb, cw1, cb1, cw2, cb2, aw, ab)
    return out.T


def _care_compact(xn, ei):
    src, dst = ei[0], ei[1]
    sim = jnp.sum(xn[src] * xn[dst], axis=1)
    perm = jnp.lexsort((-sim, src))
    src_s = src[perm]
    dst_s = dst[perm]
    nodes = jnp.arange(_N, dtype=src_s.dtype)
    starts = jnp.searchsorted(src_s, nodes, side="left")
    ends = jnp.searchsorted(src_s, nodes, side="right")
    cnt = jnp.minimum(ends - starts, _TOPK)
    kk = jnp.arange(_TOPK, dtype=starts.dtype)
    validk = kk[None, :] < cnt[:, None]
    pos = jnp.where(validk, starts[:, None] + kk[None, :], 0)
    dstk = jnp.where(validk, dst_s[pos], 0)
    deg = cnt.astype(jnp.float32)
    dinv = jnp.where(deg > 0, 1.0 / jnp.sqrt(jnp.maximum(deg, 1.0)), 0.0)
    wmat = -(dinv[:, None] * dinv[dstk]) * validk.astype(jnp.float32)
    return dstk, wmat


def _prop(h, dstk, wmat):
    contrib = wmat[:, :, None] * h[:, None, :]
    return jnp.zeros_like(h).at[dstk.reshape(-1)].add(
        contrib.reshape(-1, h.shape[1]))


def kernel(x, edge_index_r1, edge_index_r2, edge_index_r3, params):
    xn = _rownorm(x)
    eis = {"r1": edge_index_r1, "r2": edge_index_r2, "r3": edge_index_r3}
    embs = []
    for r in _RELS:
        dstk, wmat = _care_compact(xn, eis[r])
        layers = params["branches"][r]
        h_prev = x
        zeros = jnp.zeros_like(x)
        for i, layer in enumerate(layers):
            hin = x if i == 0 else h_prev
            t1 = _prop(hin, dstk, wmat)
            t2 = 2.0 * _prop(t1, dstk, wmat) - hin
            skip = h_prev if i > 0 else zeros
            h_prev = _cheb_dense(hin, t1, t2, layer["Ws"][0], layer["Ws"][1],
                                 layer["Ws"][2], layer["b"], skip)
        embs.append(h_prev)
    return _fusion(embs[0], embs[1], embs[2], params)
